# baseline (speedup 1.0000x reference)
import functools

import jax
import jax.numpy as jnp
import numpy as np
from jax import lax
from jax.experimental import pallas as pl
from jax.experimental.pallas import tpu as pltpu
from jax.experimental.pallas import tpu_sc as plsc

N = 1024
B = 16
H = 256
L = 4
G = 50
CUTOFF = 5.0
K = 32

NE = N * K
GP = 64
LANES = 16
NC, NS = 1, 16
NW = NC * NS
TPW = N // NW
CV = N // LANES

BIGF = np.float32(1e10)
LOG2 = np.float32(np.log(2.0))
EPS = np.float32(1e-12)
CUT2 = np.float32(CUTOFF * CUTOFF)
GSTEP = np.float32(CUTOFF / (G - 1))
GCOEF = np.float32(-0.5 / (CUTOFF / (G - 1)) ** 2)
PI_OVER_CUT = np.float32(np.pi / CUTOFF)

@functools.cache
def _sc_mesh():
    return plsc.VectorSubcoreMesh(
        core_axis_name="c", subcore_axis_name="s", num_cores=NC, num_subcores=NS)


def _ssp(x):
    return jnp.maximum(x, 0.0) + jnp.log1p(jnp.exp(-jnp.abs(x))) - LOG2


def _select_body(posT_hbm, batch_hbm, nbr_hbm, d2_hbm,
                 posT_v, batch_v, cand_v, oi_v, od_v, offs_s):
    wid = lax.axis_index("s") * NC + lax.axis_index("c")
    base = wid * TPW
    pltpu.sync_copy(posT_hbm, posT_v)
    pltpu.sync_copy(batch_hbm, batch_v)
    lane = lax.iota(jnp.int32, LANES)

    def count_body(j, accs):
        v = batch_v[pl.ds(j * LANES, LANES)]
        return tuple(
            acc + (v <= b).astype(jnp.int32) for b, acc in enumerate(accs))

    cnts = lax.fori_loop(
        0, CV, count_body, tuple(jnp.zeros((LANES,), jnp.int32) for _ in range(B)))
    offs_s[0] = 0
    for b in range(B):
        offs_s[b + 1] = jnp.sum(cnts[b])

    z16i = jnp.zeros((LANES,), jnp.int32)

    def target_body(t, _):
        i = base + t
        ivec = jnp.full((LANES,), i, jnp.int32)
        bi = plsc.load_gather(batch_v, [ivec])[0]
        st = offs_s[bi]
        en = offs_s[bi + 1]
        c0 = st // LANES
        c1 = (en + LANES - 1) // LANES
        xi = plsc.load_gather(posT_v, [z16i, ivec])
        yi = plsc.load_gather(posT_v, [z16i + 1, ivec])
        zi = plsc.load_gather(posT_v, [z16i + 2, ivec])

        def fill_body(j, _):
            idxv = j * LANES + lane
            xs = posT_v[0, pl.ds(j * LANES, LANES)]
            ys = posT_v[1, pl.ds(j * LANES, LANES)]
            zs = posT_v[2, pl.ds(j * LANES, LANES)]
            dx = xi - xs
            dy = yi - ys
            dz = zi - zs
            d2 = (dx * dx + dy * dy) + dz * dz
            m = (idxv >= st) & (idxv < en) & (idxv != i)
            cand_v[pl.ds(j * LANES, LANES)] = jnp.where(m, d2, BIGF)
            return 0

        lax.fori_loop(c0, c1, fill_body, 0)

        def ext_body(kk, carry):
            ilo, ihi, dlo, dhi = carry

            def scan(j, mc):
                minv, argv = mc
                v = cand_v[pl.ds(j * LANES, LANES)]
                lt = v < minv
                argv = jnp.where(lt, jnp.full((LANES,), j, jnp.int32), argv)
                minv = jnp.where(lt, v, minv)
                return minv, argv

            minv, argv = lax.fori_loop(
                c0, c1, scan,
                (jnp.full((LANES,), BIGF), jnp.zeros((LANES,), jnp.int32)))
            gm = jnp.min(minv)
            idxc = argv * LANES + lane
            sel = jnp.min(jnp.where(minv == gm, idxc, jnp.int32(N)))
            sel = jnp.minimum(sel, jnp.int32(N - 1))

            @pl.when(gm < BIGF * 0.5)
            def _clear():
                jv = sel // LANES
                ln = sel - jv * LANES
                vv = cand_v[pl.ds(jv * LANES, LANES)]
                cand_v[pl.ds(jv * LANES, LANES)] = jnp.where(lane == ln, BIGF, vv)

            mlo = (lane == kk)
            mhi = (lane == (kk - LANES))
            ilo = jnp.where(mlo, sel, ilo)
            dlo = jnp.where(mlo, gm, dlo)
            ihi = jnp.where(mhi, sel, ihi)
            dhi = jnp.where(mhi, gm, dhi)
            return ilo, ihi, dlo, dhi

        z16 = jnp.zeros((LANES,), jnp.int32)
        bf16v = jnp.full((LANES,), BIGF)
        ilo, ihi, dlo, dhi = lax.fori_loop(
            0, K, ext_body, (z16, z16, bf16v, bf16v))
        oi_v[t, pl.ds(0, LANES)] = ilo
        oi_v[t, pl.ds(LANES, LANES)] = ihi
        od_v[t, pl.ds(0, LANES)] = dlo
        od_v[t, pl.ds(LANES, LANES)] = dhi
        return 0

    lax.fori_loop(0, TPW, target_body, 0)
    pltpu.sync_copy(oi_v, nbr_hbm.at[pl.ds(base, TPW)])
    pltpu.sync_copy(od_v, d2_hbm.at[pl.ds(base, TPW)])


@functools.cache
def _build_select():
    return pl.kernel(
        _select_body,
        out_type=[
            jax.ShapeDtypeStruct((N, K), jnp.int32),
            jax.ShapeDtypeStruct((N, K), jnp.float32),
        ],
        mesh=_sc_mesh(),
        compiler_params=pltpu.CompilerParams(needs_layout_passes=False),
        scratch_types=[
            pltpu.VMEM((3, N), jnp.float32),
            pltpu.VMEM((N,), jnp.int32),
            pltpu.VMEM((N,), jnp.float32),
            pltpu.VMEM((TPW, K), jnp.int32),
            pltpu.VMEM((TPW, K), jnp.float32),
            pltpu.SMEM((B + 1,), jnp.int32),
        ],
    )


def _select(posT, batch):
    return _build_select()(posT, batch)


_GCH = 128
_GNC = NE // (NW * _GCH)


def _gather_body(table_hbm, idx_hbm, out_hbm, idx_v, rows_v, sems):
    wid = lax.axis_index("s") * NC + lax.axis_index("c")
    first = wid * _GNC * _GCH

    pltpu.sync_copy(idx_hbm.at[pl.ds(first, _GCH)], idx_v.at[0])
    g0 = pltpu.async_copy(table_hbm.at[idx_v.at[0]], rows_v.at[0], sems.at[0])
    for c in range(_GNC):
        b = c % 2
        nb = (c + 1) % 2
        if c + 1 < _GNC:
            nxt = first + (c + 1) * _GCH
            pltpu.sync_copy(idx_hbm.at[pl.ds(nxt, _GCH)], idx_v.at[nb])
            gn = pltpu.async_copy(
                table_hbm.at[idx_v.at[nb]], rows_v.at[nb], sems.at[nb])
        g0.wait()
        pltpu.sync_copy(rows_v.at[b], out_hbm.at[pl.ds(first + c * _GCH, _GCH)])
        if c + 1 < _GNC:
            g0 = gn


@functools.cache
def _build_gather():
    return pl.kernel(
        _gather_body,
        out_type=jax.ShapeDtypeStruct((NE, H), jnp.float32),
        mesh=_sc_mesh(),
        compiler_params=pltpu.CompilerParams(needs_layout_passes=False),
        scratch_types=[
            pltpu.VMEM((2, _GCH), jnp.int32),
            pltpu.VMEM((2, _GCH, H), jnp.float32),
            pltpu.SemaphoreType.DMA((2,)),
        ],
    )


def _gather(table, idx):
    return _build_gather()(table, idx)


_PB = 128


def _prep_body(z_ref, d2_ref, emb_ref, lin1_ref, h_ref, x1_ref, ea_ref, c_ref):
    zb = z_ref[...]
    oh = (zb == lax.broadcasted_iota(jnp.int32, (_PB, 128), 1)).astype(jnp.float32)
    h = jnp.dot(oh, emb_ref[...], preferred_element_type=jnp.float32)
    h_ref[...] = h
    x1_ref[...] = jnp.dot(h, lin1_ref[...], preferred_element_type=jnp.float32)

    d23 = d2_ref[...]
    ew3 = jnp.sqrt(jnp.maximum(d23, EPS))
    c_ref[...] = 0.5 * (jnp.cos(ew3 * PI_OVER_CUT) + 1.0) * (d23 <= CUT2).astype(jnp.float32)
    ewb = jnp.broadcast_to(ew3, (_PB, K, GP))
    gi = lax.broadcasted_iota(jnp.int32, (_PB, K, GP), 2)
    off = gi.astype(jnp.float32) * GSTEP
    ea = jnp.exp(GCOEF * (ewb - off) ** 2)
    ea_ref[...] = jnp.where(gi < G, ea, 0.0)


def _prep(z2, d23, embp, lin1):
    grid = N // _PB
    return pl.pallas_call(
        _prep_body,
        grid=(grid,),
        in_specs=[
            pl.BlockSpec((_PB, 1), lambda i: (i, 0)),
            pl.BlockSpec((_PB, K, 1), lambda i: (i, 0, 0)),
            pl.BlockSpec((128, H), lambda i: (0, 0)),
            pl.BlockSpec((H, H), lambda i: (0, 0)),
        ],
        out_specs=[
            pl.BlockSpec((_PB, H), lambda i: (i, 0)),
            pl.BlockSpec((_PB, H), lambda i: (i, 0)),
            pl.BlockSpec((_PB, K, GP), lambda i: (i, 0, 0)),
            pl.BlockSpec((_PB, K, 1), lambda i: (i, 0, 0)),
        ],
        out_shape=[
            jax.ShapeDtypeStruct((N, H), jnp.float32),
            jax.ShapeDtypeStruct((N, H), jnp.float32),
            jax.ShapeDtypeStruct((N, K, GP), jnp.float32),
            jax.ShapeDtypeStruct((N, K, 1), jnp.float32),
        ],
    )(z2, d23, embp, lin1)


_TB = 64
_EB = _TB * K


def _layer_body(ea_ref, c_ref, xg_ref, h_ref, w1_ref, b1_ref, w2_ref, b2_ref,
                l2w_ref, l2b_ref, lw_ref, lb_ref, l1n_ref, hn_ref, x1n_ref):
    f = _ssp(jnp.dot(ea_ref[...], w1_ref[...], preferred_element_type=jnp.float32)
             + b1_ref[...])
    wf = jnp.dot(f.astype(jnp.bfloat16), w2_ref[...].astype(jnp.bfloat16),
                 preferred_element_type=jnp.float32) + b2_ref[...]
    wf3 = wf.reshape(_TB, K, H) * c_ref[...]
    p = xg_ref[...].reshape(_TB, K, H) * wf3
    agg = jnp.sum(p, axis=1)
    x2 = jnp.dot(agg, l2w_ref[...], preferred_element_type=jnp.float32) + l2b_ref[...]
    hn = h_ref[...] + jnp.dot(_ssp(x2), lw_ref[...],
                              preferred_element_type=jnp.float32) + lb_ref[...]
    hn_ref[...] = hn
    x1n_ref[...] = jnp.dot(hn, l1n_ref[...], preferred_element_type=jnp.float32)


def _layer(ea2, c3, xg, h, w1, b1, w2, b2, l2w, l2b, lw, lb, l1n):
    grid = N // _TB
    full = lambda i: (0, 0)
    return pl.pallas_call(
        _layer_body,
        grid=(grid,),
        in_specs=[
            pl.BlockSpec((_EB, GP), lambda i: (i, 0)),
            pl.BlockSpec((_TB, K, 1), lambda i: (i, 0, 0)),
            pl.BlockSpec((_EB, H), lambda i: (i, 0)),
            pl.BlockSpec((_TB, H), lambda i: (i, 0)),
            pl.BlockSpec((GP, H), full),
            pl.BlockSpec((1, H), full),
            pl.BlockSpec((H, H), full),
            pl.BlockSpec((1, H), full),
            pl.BlockSpec((H, H), full),
            pl.BlockSpec((1, H), full),
            pl.BlockSpec((H, H), full),
            pl.BlockSpec((1, H), full),
            pl.BlockSpec((H, H), full),
        ],
        out_specs=[
            pl.BlockSpec((_TB, H), lambda i: (i, 0)),
            pl.BlockSpec((_TB, H), lambda i: (i, 0)),
        ],
        out_shape=[
            jax.ShapeDtypeStruct((N, H), jnp.float32),
            jax.ShapeDtypeStruct((N, H), jnp.float32),
        ],
    )(ea2, c3, xg, h, w1, b1, w2, b2, l2w, l2b, lw, lb, l1n)


def _pool_body(b2_ref, h_ref, pw_ref, pb_ref, out_ref):
    st = (b2_ref[...] == lax.broadcasted_iota(jnp.int32, (N, B), 1)).astype(jnp.float32)
    dn = (((0,), (0,)), ((), ()))
    sums = lax.dot_general(st, h_ref[...], dn, preferred_element_type=jnp.float32)
    cnt = lax.dot_general(st, jnp.ones((N, 1), jnp.float32), dn,
                          preferred_element_type=jnp.float32)
    pooled = jnp.where(cnt > 0, sums / jnp.maximum(cnt, 1.0), 0.0)
    out_ref[...] = (jnp.dot(pooled, pw_ref[...], preferred_element_type=jnp.float32)
                    + pb_ref[...])


def _pool(h, batch2, pw, pb2):
    return pl.pallas_call(
        _pool_body,
        out_shape=jax.ShapeDtypeStruct((B, H), jnp.float32),
    )(batch2, h, pw, pb2)


def kernel(z, pos, batch, emb, mlp_w1, mlp_b1, mlp_w2, mlp_b2,
           lin1_w, lin2_w, lin2_b, lin_w, lin_b, pool_w, pool_b):
    z = z.astype(jnp.int32)
    batch = batch.astype(jnp.int32)
    posT = pos.T.astype(jnp.float32)

    nbr, d2 = _select(posT, batch)
    nbr_flat = nbr.reshape(NE)
    d23 = d2.reshape(N, K, 1)

    embp = jnp.pad(emb, ((0, 128 - emb.shape[0]), (0, 0)))
    w1p = jnp.pad(mlp_w1, ((0, 0), (0, GP - G), (0, 0)))

    h, x1, ea, c3 = _prep(z.reshape(N, 1), d23, embp, lin1_w[0])
    ea2 = ea.reshape(NE, GP)

    for l in range(L):
        xg = _gather(x1, nbr_flat)
        h, x1 = _layer(
            ea2, c3, xg, h,
            w1p[l], mlp_b1[l].reshape(1, H), mlp_w2[l], mlp_b2[l].reshape(1, H),
            lin2_w[l], lin2_b[l].reshape(1, H), lin_w[l], lin_b[l].reshape(1, H),
            lin1_w[(l + 1) % L])

    return _pool(h, batch.reshape(N, 1), pool_w, pool_b.reshape(1, H))


# speedup vs baseline: 1.1042x; 1.1042x over previous
import functools

import jax
import jax.numpy as jnp
import numpy as np
from jax import lax
from jax.experimental import pallas as pl
from jax.experimental.pallas import tpu as pltpu
from jax.experimental.pallas import tpu_sc as plsc

N = 1024
B = 16
H = 256
L = 4
G = 50
CUTOFF = 5.0
K = 32

NE = N * K
GP = 64
LANES = 16
NC, NS = 2, 16
NW = NC * NS
TPW = N // NW
CV = N // LANES

BIGF = np.float32(1e10)
LOG2 = np.float32(np.log(2.0))
EPS = np.float32(1e-12)
CUT2 = np.float32(CUTOFF * CUTOFF)
GSTEP = np.float32(CUTOFF / (G - 1))
GCOEF = np.float32(-0.5 / (CUTOFF / (G - 1)) ** 2)
PI_OVER_CUT = np.float32(np.pi / CUTOFF)

@functools.cache
def _sc_mesh():
    return plsc.VectorSubcoreMesh(
        core_axis_name="c", subcore_axis_name="s", num_cores=NC, num_subcores=NS)


def _ssp(x):
    return jnp.maximum(x, 0.0) + jnp.log1p(jnp.exp(-jnp.abs(x))) - LOG2


def _select_body(posT_hbm, batch_hbm, nbr_hbm, d2_hbm,
                 posT_v, batch_v, cand_v, oi_v, od_v, offs_s):
    wid = lax.axis_index("s") * NC + lax.axis_index("c")
    base = wid * TPW
    pltpu.sync_copy(posT_hbm, posT_v)
    pltpu.sync_copy(batch_hbm, batch_v)
    lane = lax.iota(jnp.int32, LANES)

    def count_body(j, accs):
        v = batch_v[pl.ds(j * LANES, LANES)]
        return tuple(
            acc + (v <= b).astype(jnp.int32) for b, acc in enumerate(accs))

    cnts = lax.fori_loop(
        0, CV, count_body, tuple(jnp.zeros((LANES,), jnp.int32) for _ in range(B)))
    offs_s[0] = 0
    for b in range(B):
        offs_s[b + 1] = jnp.sum(cnts[b])

    z16i = jnp.zeros((LANES,), jnp.int32)

    def target_body(t, _):
        i = base + t
        ivec = jnp.full((LANES,), i, jnp.int32)
        bi = plsc.load_gather(batch_v, [ivec])[0]
        st = offs_s[bi]
        en = offs_s[bi + 1]
        c0 = st // LANES
        c1 = (en + LANES - 1) // LANES
        xi = plsc.load_gather(posT_v, [z16i, ivec])
        yi = plsc.load_gather(posT_v, [z16i + 1, ivec])
        zi = plsc.load_gather(posT_v, [z16i + 2, ivec])

        def fill_body(j, _):
            idxv = j * LANES + lane
            xs = posT_v[0, pl.ds(j * LANES, LANES)]
            ys = posT_v[1, pl.ds(j * LANES, LANES)]
            zs = posT_v[2, pl.ds(j * LANES, LANES)]
            dx = xi - xs
            dy = yi - ys
            dz = zi - zs
            d2 = (dx * dx + dy * dy) + dz * dz
            m = (idxv >= st) & (idxv < en) & (idxv != i)
            cand_v[pl.ds(j * LANES, LANES)] = jnp.where(m, d2, BIGF)
            return 0

        lax.fori_loop(c0, c1, fill_body, 0)

        def ext_body(kk, carry):
            ilo, ihi, dlo, dhi = carry

            def scan(j, mc):
                minv, argv = mc
                v = cand_v[pl.ds(j * LANES, LANES)]
                lt = v < minv
                argv = jnp.where(lt, jnp.full((LANES,), j, jnp.int32), argv)
                minv = jnp.where(lt, v, minv)
                return minv, argv

            minv, argv = lax.fori_loop(
                c0, c1, scan,
                (jnp.full((LANES,), BIGF), jnp.zeros((LANES,), jnp.int32)))
            gm = jnp.min(minv)
            idxc = argv * LANES + lane
            sel = jnp.min(jnp.where(minv == gm, idxc, jnp.int32(N)))
            sel = jnp.minimum(sel, jnp.int32(N - 1))

            @pl.when(gm < BIGF * 0.5)
            def _clear():
                jv = sel // LANES
                ln = sel - jv * LANES
                vv = cand_v[pl.ds(jv * LANES, LANES)]
                cand_v[pl.ds(jv * LANES, LANES)] = jnp.where(lane == ln, BIGF, vv)

            mlo = (lane == kk)
            mhi = (lane == (kk - LANES))
            ilo = jnp.where(mlo, sel, ilo)
            dlo = jnp.where(mlo, gm, dlo)
            ihi = jnp.where(mhi, sel, ihi)
            dhi = jnp.where(mhi, gm, dhi)
            return ilo, ihi, dlo, dhi

        z16 = jnp.zeros((LANES,), jnp.int32)
        bf16v = jnp.full((LANES,), BIGF)
        ilo, ihi, dlo, dhi = lax.fori_loop(
            0, K, ext_body, (z16, z16, bf16v, bf16v))
        oi_v[t, pl.ds(0, LANES)] = ilo
        oi_v[t, pl.ds(LANES, LANES)] = ihi
        od_v[t, pl.ds(0, LANES)] = dlo
        od_v[t, pl.ds(LANES, LANES)] = dhi
        return 0

    lax.fori_loop(0, TPW, target_body, 0)
    pltpu.sync_copy(oi_v, nbr_hbm.at[pl.ds(base, TPW)])
    pltpu.sync_copy(od_v, d2_hbm.at[pl.ds(base, TPW)])


@functools.cache
def _build_select():
    return pl.kernel(
        _select_body,
        out_type=[
            jax.ShapeDtypeStruct((N, K), jnp.int32),
            jax.ShapeDtypeStruct((N, K), jnp.float32),
        ],
        mesh=_sc_mesh(),
        compiler_params=pltpu.CompilerParams(needs_layout_passes=False),
        scratch_types=[
            pltpu.VMEM((3, N), jnp.float32),
            pltpu.VMEM((N,), jnp.int32),
            pltpu.VMEM((N,), jnp.float32),
            pltpu.VMEM((TPW, K), jnp.int32),
            pltpu.VMEM((TPW, K), jnp.float32),
            pltpu.SMEM((B + 1,), jnp.int32),
        ],
    )


def _select(posT, batch):
    return _build_select()(posT, batch)


_GCH = 128


def _gather_body(gnc, table_hbm, idx_hbm, out_hbm, idx_v, rows_v, sems):
    wid = lax.axis_index("s") * NC + lax.axis_index("c")
    first = wid * gnc * _GCH

    pltpu.sync_copy(idx_hbm.at[pl.ds(first, _GCH)], idx_v.at[0])
    g0 = pltpu.async_copy(table_hbm.at[idx_v.at[0]], rows_v.at[0], sems.at[0])
    for c in range(gnc):
        b = c % 2
        nb = (c + 1) % 2
        if c + 1 < gnc:
            nxt = first + (c + 1) * _GCH
            pltpu.sync_copy(idx_hbm.at[pl.ds(nxt, _GCH)], idx_v.at[nb])
            gn = pltpu.async_copy(
                table_hbm.at[idx_v.at[nb]], rows_v.at[nb], sems.at[nb])
        g0.wait()
        pltpu.sync_copy(rows_v.at[b], out_hbm.at[pl.ds(first + c * _GCH, _GCH)])
        if c + 1 < gnc:
            g0 = gn


@functools.cache
def _build_gather(nrows):
    gnc = nrows // (NW * _GCH)
    return pl.kernel(
        functools.partial(_gather_body, gnc),
        out_type=jax.ShapeDtypeStruct((nrows, H), jnp.float32),
        mesh=_sc_mesh(),
        compiler_params=pltpu.CompilerParams(needs_layout_passes=False),
        scratch_types=[
            pltpu.VMEM((2, _GCH), jnp.int32),
            pltpu.VMEM((2, _GCH, H), jnp.float32),
            pltpu.SemaphoreType.DMA((2,)),
        ],
    )


def _gather(table, idx):
    return _build_gather(idx.shape[0])(table, idx)


_PB = 128


def _emb_body(z_ref, emb_ref, lin1_ref, h_ref, x1_ref):
    zb = z_ref[...]
    oh = (zb == lax.broadcasted_iota(jnp.int32, (_PB, 128), 1)).astype(jnp.float32)
    h = jnp.dot(oh, emb_ref[...], preferred_element_type=jnp.float32)
    h_ref[...] = h
    x1_ref[...] = jnp.dot(h, lin1_ref[...], preferred_element_type=jnp.float32)


def _emb(z2, embp, lin1):
    return pl.pallas_call(
        _emb_body,
        grid=(N // _PB,),
        in_specs=[
            pl.BlockSpec((_PB, 1), lambda i: (i, 0)),
            pl.BlockSpec((128, H), lambda i: (0, 0)),
            pl.BlockSpec((H, H), lambda i: (0, 0)),
        ],
        out_specs=[
            pl.BlockSpec((_PB, H), lambda i: (i, 0)),
            pl.BlockSpec((_PB, H), lambda i: (i, 0)),
        ],
        out_shape=[
            jax.ShapeDtypeStruct((N, H), jnp.float32),
            jax.ShapeDtypeStruct((N, H), jnp.float32),
        ],
    )(z2, embp, lin1)


def _edge_body(d2_ref, ea_ref, c_ref):
    d23 = d2_ref[...]
    ew3 = jnp.sqrt(jnp.maximum(d23, EPS))
    c_ref[...] = 0.5 * (jnp.cos(ew3 * PI_OVER_CUT) + 1.0) * (d23 <= CUT2).astype(jnp.float32)
    ewb = jnp.broadcast_to(ew3, (_PB, K, GP))
    gi = lax.broadcasted_iota(jnp.int32, (_PB, K, GP), 2)
    off = gi.astype(jnp.float32) * GSTEP
    ea = jnp.exp(GCOEF * (ewb - off) ** 2)
    ea_ref[...] = jnp.where(gi < G, ea, 0.0)


def _edges(d23):
    return pl.pallas_call(
        _edge_body,
        grid=(N // _PB,),
        in_specs=[pl.BlockSpec((_PB, K, 1), lambda i: (i, 0, 0))],
        out_specs=[
            pl.BlockSpec((_PB, K, GP), lambda i: (i, 0, 0)),
            pl.BlockSpec((_PB, K, 1), lambda i: (i, 0, 0)),
        ],
        out_shape=[
            jax.ShapeDtypeStruct((N, K, GP), jnp.float32),
            jax.ShapeDtypeStruct((N, K, 1), jnp.float32),
        ],
    )(d23)


_TB = 64
_EB = _TB * K


def _layer_body(ea_ref, c_ref, xg_ref, h_ref, w1_ref, b1_ref, w2_ref, b2_ref,
                l2w_ref, l2b_ref, lw_ref, lb_ref, l1n_ref, hn_ref, x1n_ref):
    f = _ssp(jnp.dot(ea_ref[...], w1_ref[...], preferred_element_type=jnp.float32)
             + b1_ref[...])
    wf = jnp.dot(f.astype(jnp.bfloat16), w2_ref[...].astype(jnp.bfloat16),
                 preferred_element_type=jnp.float32) + b2_ref[...]
    wf3 = wf.reshape(_TB, K, H) * c_ref[...]
    p = xg_ref[...].reshape(_TB, K, H) * wf3
    agg = jnp.sum(p, axis=1)
    x2 = jnp.dot(agg, l2w_ref[...], preferred_element_type=jnp.float32) + l2b_ref[...]
    hn = h_ref[...] + jnp.dot(_ssp(x2), lw_ref[...],
                              preferred_element_type=jnp.float32) + lb_ref[...]
    hn_ref[...] = hn
    x1n_ref[...] = jnp.dot(hn, l1n_ref[...], preferred_element_type=jnp.float32)


def _layer(ea2, c3, xg, h, w1, b1, w2, b2, l2w, l2b, lw, lb, l1n):
    rows = h.shape[0]
    grid = rows // _TB
    full = lambda i: (0, 0)
    return pl.pallas_call(
        _layer_body,
        grid=(grid,),
        in_specs=[
            pl.BlockSpec((_EB, GP), lambda i: (i, 0)),
            pl.BlockSpec((_TB, K, 1), lambda i: (i, 0, 0)),
            pl.BlockSpec((_EB, H), lambda i: (i, 0)),
            pl.BlockSpec((_TB, H), lambda i: (i, 0)),
            pl.BlockSpec((GP, H), full),
            pl.BlockSpec((1, H), full),
            pl.BlockSpec((H, H), full),
            pl.BlockSpec((1, H), full),
            pl.BlockSpec((H, H), full),
            pl.BlockSpec((1, H), full),
            pl.BlockSpec((H, H), full),
            pl.BlockSpec((1, H), full),
            pl.BlockSpec((H, H), full),
        ],
        out_specs=[
            pl.BlockSpec((_TB, H), lambda i: (i, 0)),
            pl.BlockSpec((_TB, H), lambda i: (i, 0)),
        ],
        out_shape=[
            jax.ShapeDtypeStruct((rows, H), jnp.float32),
            jax.ShapeDtypeStruct((rows, H), jnp.float32),
        ],
    )(ea2, c3, xg, h, w1, b1, w2, b2, l2w, l2b, lw, lb, l1n)


def _pool_body(b2_ref, h_ref, pw_ref, pb_ref, out_ref):
    st = (b2_ref[...] == lax.broadcasted_iota(jnp.int32, (N, B), 1)).astype(jnp.float32)
    dn = (((0,), (0,)), ((), ()))
    sums = lax.dot_general(st, h_ref[...], dn, preferred_element_type=jnp.float32)
    cnt = lax.dot_general(st, jnp.ones((N, 1), jnp.float32), dn,
                          preferred_element_type=jnp.float32)
    pooled = jnp.where(cnt > 0, sums / jnp.maximum(cnt, 1.0), 0.0)
    out_ref[...] = (jnp.dot(pooled, pw_ref[...], preferred_element_type=jnp.float32)
                    + pb_ref[...])


def _pool(h, batch2, pw, pb2):
    return pl.pallas_call(
        _pool_body,
        out_shape=jax.ShapeDtypeStruct((B, H), jnp.float32),
    )(batch2, h, pw, pb2)


def kernel(z, pos, batch, emb, mlp_w1, mlp_b1, mlp_w2, mlp_b2,
           lin1_w, lin2_w, lin2_b, lin_w, lin_b, pool_w, pool_b):
    z = z.astype(jnp.int32)
    batch = batch.astype(jnp.int32)
    posT = pos.T.astype(jnp.float32)

    nbr, d2 = _select(posT, batch)
    nbr_flat = nbr.reshape(NE)
    d23 = d2.reshape(N, K, 1)

    embp = jnp.pad(emb, ((0, 128 - emb.shape[0]), (0, 0)))
    w1p = jnp.pad(mlp_w1, ((0, 0), (0, GP - G), (0, 0)))

    h, x1 = _emb(z.reshape(N, 1), embp, lin1_w[0])
    ea, c3 = _edges(d23)
    ea2 = ea.reshape(NE, GP)

    NH = N // 2
    EH = NE // 2
    for l in range(L):
        args = (w1p[l], mlp_b1[l].reshape(1, H), mlp_w2[l],
                mlp_b2[l].reshape(1, H), lin2_w[l], lin2_b[l].reshape(1, H),
                lin_w[l], lin_b[l].reshape(1, H), lin1_w[(l + 1) % L])
        xga = _gather(x1, nbr_flat[:EH])
        xgb = _gather(x1, nbr_flat[EH:])
        ha, x1a = _layer(ea2[:EH], c3[:NH], xga, h[:NH], *args)
        hb, x1b = _layer(ea2[EH:], c3[NH:], xgb, h[NH:], *args)
        h = jnp.concatenate([ha, hb])
        x1 = jnp.concatenate([x1a, x1b])

    return _pool(h, batch.reshape(N, 1), pool_w, pool_b.reshape(1, H))


# speedup vs baseline: 1.4642x; 1.3261x over previous
import functools

import jax
import jax.numpy as jnp
import numpy as np
from jax import lax
from jax.experimental import pallas as pl
from jax.experimental.pallas import tpu as pltpu
from jax.experimental.pallas import tpu_sc as plsc

N = 1024
B = 16
H = 256
L = 4
G = 50
CUTOFF = 5.0
K = 32

NE = N * K
GP = 64
LANES = 16
NC, NS = 2, 16
NW = NC * NS
TPW = N // NW
CV = N // LANES

BIGF = np.float32(1e10)
LOG2 = np.float32(np.log(2.0))
EPS = np.float32(1e-12)
CUT2 = np.float32(CUTOFF * CUTOFF)
GSTEP = np.float32(CUTOFF / (G - 1))
GCOEF = np.float32(-0.5 / (CUTOFF / (G - 1)) ** 2)
PI_OVER_CUT = np.float32(np.pi / CUTOFF)

@functools.cache
def _sc_mesh():
    return plsc.VectorSubcoreMesh(
        core_axis_name="c", subcore_axis_name="s", num_cores=NC, num_subcores=NS)


def _ssp(x):
    return jnp.maximum(x, 0.0) + jnp.log1p(jnp.exp(-jnp.abs(x))) - LOG2


def _select_body(posT_hbm, batch_hbm, nbr_hbm, d2_hbm,
                 posT_v, batch_v, cand_v, oi_v, od_v, offs_s):
    wid = lax.axis_index("s") * NC + lax.axis_index("c")
    base = wid * TPW
    pltpu.sync_copy(posT_hbm, posT_v)
    pltpu.sync_copy(batch_hbm, batch_v)
    lane = lax.iota(jnp.int32, LANES)

    def count_body(j, accs):
        v = batch_v[pl.ds(j * LANES, LANES)]
        return tuple(
            acc + (v <= b).astype(jnp.int32) for b, acc in enumerate(accs))

    cnts = lax.fori_loop(
        0, CV, count_body, tuple(jnp.zeros((LANES,), jnp.int32) for _ in range(B)))
    offs_s[0] = 0
    for b in range(B):
        offs_s[b + 1] = jnp.sum(cnts[b])

    z16i = jnp.zeros((LANES,), jnp.int32)

    def target_body(t, _):
        i = base + t
        ivec = jnp.full((LANES,), i, jnp.int32)
        bi = plsc.load_gather(batch_v, [ivec])[0]
        st = offs_s[bi]
        en = offs_s[bi + 1]
        c0 = st // LANES
        c1 = (en + LANES - 1) // LANES
        xi = plsc.load_gather(posT_v, [z16i, ivec])
        yi = plsc.load_gather(posT_v, [z16i + 1, ivec])
        zi = plsc.load_gather(posT_v, [z16i + 2, ivec])

        def merge(ak, ai, bk, bi):
            rk = lax.rev(bk, (0,))
            ri = lax.rev(bi, (0,))
            lt = ak < rk
            lok = jnp.where(lt, ak, rk)
            loi = jnp.where(lt, ai, ri)
            hik = jnp.where(lt, rk, ak)
            hii = jnp.where(lt, ri, ai)
            lok, loi = plsc.sort_key_val(lok, loi)
            hik, hii = plsc.sort_key_val(hik, hii)
            return lok, loi, hik, hii

        def vreg_body(j, carry):
            t0k, t0i, t1k, t1i = carry
            idxv = j * LANES + lane
            xs = posT_v[0, pl.ds(j * LANES, LANES)]
            ys = posT_v[1, pl.ds(j * LANES, LANES)]
            zs = posT_v[2, pl.ds(j * LANES, LANES)]
            dx = xi - xs
            dy = yi - ys
            dz = zi - zs
            d2 = (dx * dx + dy * dy) + dz * dz
            m = (idxv >= st) & (idxv < en) & (idxv != i)
            xk, xi_ = plsc.sort_key_val(jnp.where(m, d2, BIGF), idxv)
            a0k, a0i, a1k, a1i = merge(t0k, t0i, xk, xi_)
            b0k, b0i, _, _ = merge(a1k, a1i, t1k, t1i)
            return a0k, a0i, b0k, b0i

        z16 = jnp.zeros((LANES,), jnp.int32)
        bf16v = jnp.full((LANES,), BIGF)
        t0k, t0i, t1k, t1i = lax.fori_loop(
            c0, c1, vreg_body, (bf16v, z16, bf16v, z16))
        oi_v[t, pl.ds(0, LANES)] = t0i
        oi_v[t, pl.ds(LANES, LANES)] = t1i
        od_v[t, pl.ds(0, LANES)] = t0k
        od_v[t, pl.ds(LANES, LANES)] = t1k
        return 0

    lax.fori_loop(0, TPW, target_body, 0)
    pltpu.sync_copy(oi_v, nbr_hbm.at[pl.ds(base, TPW)])
    pltpu.sync_copy(od_v, d2_hbm.at[pl.ds(base, TPW)])


@functools.cache
def _build_select():
    return pl.kernel(
        _select_body,
        out_type=[
            jax.ShapeDtypeStruct((N, K), jnp.int32),
            jax.ShapeDtypeStruct((N, K), jnp.float32),
        ],
        mesh=_sc_mesh(),
        compiler_params=pltpu.CompilerParams(needs_layout_passes=False),
        scratch_types=[
            pltpu.VMEM((3, N), jnp.float32),
            pltpu.VMEM((N,), jnp.int32),
            pltpu.VMEM((N,), jnp.float32),
            pltpu.VMEM((TPW, K), jnp.int32),
            pltpu.VMEM((TPW, K), jnp.float32),
            pltpu.SMEM((B + 1,), jnp.int32),
        ],
    )


def _select(posT, batch):
    return _build_select()(posT, batch)


_GCH = 128


def _gather_body(gnc, table_hbm, idx_hbm, out_hbm, idx_v, rows_v, sems):
    wid = lax.axis_index("s") * NC + lax.axis_index("c")
    first = wid * gnc * _GCH

    pltpu.sync_copy(idx_hbm.at[pl.ds(first, _GCH)], idx_v.at[0])
    g0 = pltpu.async_copy(table_hbm.at[idx_v.at[0]], rows_v.at[0], sems.at[0])
    for c in range(gnc):
        b = c % 2
        nb = (c + 1) % 2
        if c + 1 < gnc:
            nxt = first + (c + 1) * _GCH
            pltpu.sync_copy(idx_hbm.at[pl.ds(nxt, _GCH)], idx_v.at[nb])
            gn = pltpu.async_copy(
                table_hbm.at[idx_v.at[nb]], rows_v.at[nb], sems.at[nb])
        g0.wait()
        pltpu.sync_copy(rows_v.at[b], out_hbm.at[pl.ds(first + c * _GCH, _GCH)])
        if c + 1 < gnc:
            g0 = gn


@functools.cache
def _build_gather(nrows):
    gnc = nrows // (NW * _GCH)
    return pl.kernel(
        functools.partial(_gather_body, gnc),
        out_type=jax.ShapeDtypeStruct((nrows, H), jnp.float32),
        mesh=_sc_mesh(),
        compiler_params=pltpu.CompilerParams(needs_layout_passes=False),
        scratch_types=[
            pltpu.VMEM((2, _GCH), jnp.int32),
            pltpu.VMEM((2, _GCH, H), jnp.float32),
            pltpu.SemaphoreType.DMA((2,)),
        ],
    )


def _gather(table, idx):
    return _build_gather(idx.shape[0])(table, idx)


_PB = 128


def _emb_body(z_ref, emb_ref, lin1_ref, h_ref, x1_ref):
    zb = z_ref[...]
    oh = (zb == lax.broadcasted_iota(jnp.int32, (_PB, 128), 1)).astype(jnp.float32)
    h = jnp.dot(oh, emb_ref[...], preferred_element_type=jnp.float32)
    h_ref[...] = h
    x1_ref[...] = jnp.dot(h, lin1_ref[...], preferred_element_type=jnp.float32)


def _emb(z2, embp, lin1):
    return pl.pallas_call(
        _emb_body,
        grid=(N // _PB,),
        in_specs=[
            pl.BlockSpec((_PB, 1), lambda i: (i, 0)),
            pl.BlockSpec((128, H), lambda i: (0, 0)),
            pl.BlockSpec((H, H), lambda i: (0, 0)),
        ],
        out_specs=[
            pl.BlockSpec((_PB, H), lambda i: (i, 0)),
            pl.BlockSpec((_PB, H), lambda i: (i, 0)),
        ],
        out_shape=[
            jax.ShapeDtypeStruct((N, H), jnp.float32),
            jax.ShapeDtypeStruct((N, H), jnp.float32),
        ],
    )(z2, embp, lin1)


def _edge_body(d2_ref, ea_ref, c_ref):
    d23 = d2_ref[...]
    ew3 = jnp.sqrt(jnp.maximum(d23, EPS))
    c_ref[...] = 0.5 * (jnp.cos(ew3 * PI_OVER_CUT) + 1.0) * (d23 <= CUT2).astype(jnp.float32)
    ewb = jnp.broadcast_to(ew3, (_PB, K, GP))
    gi = lax.broadcasted_iota(jnp.int32, (_PB, K, GP), 2)
    off = gi.astype(jnp.float32) * GSTEP
    ea = jnp.exp(GCOEF * (ewb - off) ** 2)
    ea_ref[...] = jnp.where(gi < G, ea, 0.0)


def _edges(d23):
    return pl.pallas_call(
        _edge_body,
        grid=(N // _PB,),
        in_specs=[pl.BlockSpec((_PB, K, 1), lambda i: (i, 0, 0))],
        out_specs=[
            pl.BlockSpec((_PB, K, GP), lambda i: (i, 0, 0)),
            pl.BlockSpec((_PB, K, 1), lambda i: (i, 0, 0)),
        ],
        out_shape=[
            jax.ShapeDtypeStruct((N, K, GP), jnp.float32),
            jax.ShapeDtypeStruct((N, K, 1), jnp.float32),
        ],
    )(d23)


_TB = 64
_EB = _TB * K


def _layer_body(ea_ref, c_ref, xg_ref, h_ref, w1_ref, b1_ref, w2_ref, b2_ref,
                l2w_ref, l2b_ref, lw_ref, lb_ref, l1n_ref, hn_ref, x1n_ref):
    f = _ssp(jnp.dot(ea_ref[...], w1_ref[...], preferred_element_type=jnp.float32)
             + b1_ref[...])
    wf = jnp.dot(f.astype(jnp.bfloat16), w2_ref[...].astype(jnp.bfloat16),
                 preferred_element_type=jnp.float32) + b2_ref[...]
    wf3 = wf.reshape(_TB, K, H) * c_ref[...]
    p = xg_ref[...].reshape(_TB, K, H) * wf3
    agg = jnp.sum(p, axis=1)
    x2 = jnp.dot(agg, l2w_ref[...], preferred_element_type=jnp.float32) + l2b_ref[...]
    hn = h_ref[...] + jnp.dot(_ssp(x2), lw_ref[...],
                              preferred_element_type=jnp.float32) + lb_ref[...]
    hn_ref[...] = hn
    x1n_ref[...] = jnp.dot(hn, l1n_ref[...], preferred_element_type=jnp.float32)


def _layer(ea2, c3, xg, h, w1, b1, w2, b2, l2w, l2b, lw, lb, l1n):
    rows = h.shape[0]
    grid = rows // _TB
    full = lambda i: (0, 0)
    return pl.pallas_call(
        _layer_body,
        grid=(grid,),
        in_specs=[
            pl.BlockSpec((_EB, GP), lambda i: (i, 0)),
            pl.BlockSpec((_TB, K, 1), lambda i: (i, 0, 0)),
            pl.BlockSpec((_EB, H), lambda i: (i, 0)),
            pl.BlockSpec((_TB, H), lambda i: (i, 0)),
            pl.BlockSpec((GP, H), full),
            pl.BlockSpec((1, H), full),
            pl.BlockSpec((H, H), full),
            pl.BlockSpec((1, H), full),
            pl.BlockSpec((H, H), full),
            pl.BlockSpec((1, H), full),
            pl.BlockSpec((H, H), full),
            pl.BlockSpec((1, H), full),
            pl.BlockSpec((H, H), full),
        ],
        out_specs=[
            pl.BlockSpec((_TB, H), lambda i: (i, 0)),
            pl.BlockSpec((_TB, H), lambda i: (i, 0)),
        ],
        out_shape=[
            jax.ShapeDtypeStruct((rows, H), jnp.float32),
            jax.ShapeDtypeStruct((rows, H), jnp.float32),
        ],
    )(ea2, c3, xg, h, w1, b1, w2, b2, l2w, l2b, lw, lb, l1n)


def _pool_body(b2_ref, h_ref, pw_ref, pb_ref, out_ref):
    st = (b2_ref[...] == lax.broadcasted_iota(jnp.int32, (N, B), 1)).astype(jnp.float32)
    dn = (((0,), (0,)), ((), ()))
    sums = lax.dot_general(st, h_ref[...], dn, preferred_element_type=jnp.float32)
    cnt = lax.dot_general(st, jnp.ones((N, 1), jnp.float32), dn,
                          preferred_element_type=jnp.float32)
    pooled = jnp.where(cnt > 0, sums / jnp.maximum(cnt, 1.0), 0.0)
    out_ref[...] = (jnp.dot(pooled, pw_ref[...], preferred_element_type=jnp.float32)
                    + pb_ref[...])


def _pool(h, batch2, pw, pb2):
    return pl.pallas_call(
        _pool_body,
        out_shape=jax.ShapeDtypeStruct((B, H), jnp.float32),
    )(batch2, h, pw, pb2)


def kernel(z, pos, batch, emb, mlp_w1, mlp_b1, mlp_w2, mlp_b2,
           lin1_w, lin2_w, lin2_b, lin_w, lin_b, pool_w, pool_b):
    z = z.astype(jnp.int32)
    batch = batch.astype(jnp.int32)
    posT = pos.T.astype(jnp.float32)

    nbr, d2 = _select(posT, batch)
    nbr_flat = nbr.reshape(NE)
    d23 = d2.reshape(N, K, 1)

    embp = jnp.pad(emb, ((0, 128 - emb.shape[0]), (0, 0)))
    w1p = jnp.pad(mlp_w1, ((0, 0), (0, GP - G), (0, 0)))

    h, x1 = _emb(z.reshape(N, 1), embp, lin1_w[0])
    ea, c3 = _edges(d23)
    ea2 = ea.reshape(NE, GP)

    for l in range(L):
        args = (w1p[l], mlp_b1[l].reshape(1, H), mlp_w2[l],
                mlp_b2[l].reshape(1, H), lin2_w[l], lin2_b[l].reshape(1, H),
                lin_w[l], lin_b[l].reshape(1, H), lin1_w[(l + 1) % L])
        xg = _gather(x1, nbr_flat)
        h, x1 = _layer(ea2, c3, xg, h, *args)

    return _pool(h, batch.reshape(N, 1), pool_w, pool_b.reshape(1, H))
